# baseline (speedup 1.0000x reference)
import functools

import jax
import jax.numpy as jnp
from jax import lax
from jax.experimental import pallas as pl
from jax.experimental.pallas import tpu as pltpu
from jax.experimental.pallas import tpu_sc as plsc

B = 16384
D = 64
L = 16
NC = 2
NS = 16
NW = NC * NS
BPW = B // NW
C = 256
NCHUNK = BPW // C
NGRP = C // L
DG = D // L


def _sc_body(h_hbm, t_hbm, r_hbm, ent_hbm, rel_hbm, res_hbm,
             hv, tv, rv, eh, et, rc, resc, sem):
    wid = lax.axis_index("s") * NC + lax.axis_index("c")
    row_ids = lax.iota(jnp.int32, L)

    for chunk in range(NCHUNK):
        base = wid * BPW + chunk * C
        pltpu.sync_copy(h_hbm.at[pl.ds(base, C)], hv)
        pltpu.sync_copy(t_hbm.at[pl.ds(base, C)], tv)
        pltpu.sync_copy(r_hbm.at[pl.ds(base, C)], rv)

        cps = [
            pltpu.async_copy(ent_hbm.at[hv], eh, sem),
            pltpu.async_copy(ent_hbm.at[tv], et, sem),
            pltpu.async_copy(rel_hbm.at[rv], rc, sem),
        ]
        for cp in cps:
            cp.wait()

        def grp_body(g, _):
            res_v = jnp.zeros((L,), jnp.float32)
            for e in range(L):
                eb = g * L + e
                acc = jnp.zeros((L,), jnp.float32)
                for dg in range(DG):
                    sl1 = pl.ds(dg * L, L)
                    sl2 = pl.ds(D + dg * L, L)
                    a1 = eh[eb, sl1]
                    a2 = eh[eb, sl2]
                    b1 = et[eb, sl1]
                    b2 = et[eb, sl2]
                    q1 = rc[eb, sl1]
                    q2 = rc[eb, sl2]
                    acc = acc + q1 * (a1 * b1 + a2 * b2) + q2 * (a1 * b2 - a2 * b1)
                s = jnp.sum(acc)
                res_v = jnp.where(row_ids == e, s, res_v)
            resc[pl.ds(g * L, L)] = res_v
            return 0

        lax.fori_loop(0, NGRP, grp_body, 0)
        pltpu.sync_copy(resc, res_hbm.at[pl.ds(base, C)])


def _make_sc_kernel():
    mesh = plsc.VectorSubcoreMesh(core_axis_name="c", subcore_axis_name="s")
    return pl.kernel(
        _sc_body,
        out_type=jax.ShapeDtypeStruct((B,), jnp.float32),
        mesh=mesh,
        compiler_params=pltpu.CompilerParams(
            needs_layout_passes=False, use_tc_tiling_on_sc=True),
        scratch_types=[
            pltpu.VMEM((C,), jnp.int32),
            pltpu.VMEM((C,), jnp.int32),
            pltpu.VMEM((C,), jnp.int32),
            pltpu.VMEM((C, 2 * D), jnp.float32),
            pltpu.VMEM((C, 2 * D), jnp.float32),
            pltpu.VMEM((C, 2 * D), jnp.float32),
            pltpu.VMEM((C,), jnp.float32),
            pltpu.SemaphoreType.DMA,
        ],
    )


def _loss_body(res_ref, y_ref, out_ref):
    x = -y_ref[...] * res_ref[...]
    out_ref[0, 0] = jnp.mean(jax.nn.softplus(x))


@jax.jit
def kernel(h, t, r, y, ent1, ent2, rel1, rel2):
    h = h.astype(jnp.int32)
    t = t.astype(jnp.int32)
    r = r.astype(jnp.int32)
    ent = jnp.concatenate([ent1, ent2], axis=1)
    rel = jnp.concatenate([rel1, rel2], axis=1)
    res = _make_sc_kernel()(h, t, r, ent, rel)
    loss = pl.pallas_call(
        _loss_body,
        out_shape=jax.ShapeDtypeStruct((1, 1), jnp.float32),
        out_specs=pl.BlockSpec(memory_space=pltpu.SMEM),
    )(res.reshape(128, 128), y.reshape(128, 128))
    return loss[0, 0]
